# baseline (speedup 1.0000x reference)
import functools

import jax
import jax.numpy as jnp
from jax.experimental import pallas as pl
from jax.experimental.pallas import tpu as pltpu


def _pool_body(*refs, CB, S):
    x_refs, o_ref = refs[:S], refs[S]
    cb = pl.program_id(1)
    acc = jnp.sum(x_refs[0][...], axis=(0, 2, 3))
    for q in range(1, S):
        acc = acc + jnp.sum(x_refs[q][...], axis=(0, 2, 3))
    o_ref[0, 0, pl.ds(cb * CB, CB)] = acc


def _rank_body(w_ref, sums_ref, idx_ref, *, B, C, k, HW):
    w0 = w_ref[0]
    w1 = w_ref[1]
    w2 = w_ref[2]
    pooled = sums_ref[:, 0, :] / jnp.float32(HW)
    zero = jnp.zeros((B, 1), jnp.float32)
    left = jnp.concatenate([zero, pooled[:, :-1]], axis=1)
    right = jnp.concatenate([pooled[:, 1:], zero], axis=1)
    conv = w0 * left + w1 * pooled + w2 * right
    s = jax.nn.sigmoid(conv)
    ii = jax.lax.broadcasted_iota(jnp.int32, (C, C), 0)
    jj = jax.lax.broadcasted_iota(jnp.int32, (C, C), 1)
    rr = jax.lax.broadcasted_iota(jnp.int32, (C, k), 1)
    ic = jax.lax.broadcasted_iota(jnp.int32, (C, k), 0)
    for b in range(B):
        u = s[b].reshape(C, 1)
        v = s[b].reshape(1, C)
        before = (v > u) | ((v == u) & (jj < ii))
        rank = jnp.sum(before.astype(jnp.int32), axis=1, keepdims=True)
        sel = (rank == rr).astype(jnp.int32)
        idx_ref[b, :] = jnp.sum(ic * sel, axis=0)


def _gather_body(*refs, U):
    x_refs, o_ref = refs[1:1 + U], refs[1 + U]
    for u in range(U):
        o_ref[0, u] = x_refs[u][0, 0]


@jax.jit
def kernel(x, conv_w):
    B, C, H, W = x.shape
    k = int(C * 0.5)
    HW = H * W

    CB = 128
    NCB = C // CB
    S = 4
    HS = H // S
    sums3 = pl.pallas_call(
        functools.partial(_pool_body, CB=CB, S=S),
        grid=(B, NCB),
        in_specs=[
            pl.BlockSpec((1, CB, HS, W), functools.partial(
                lambda q, b, cb: (b, cb, q, 0), q))
            for q in range(S)
        ],
        out_specs=pl.BlockSpec((1, 1, C), lambda b, cb: (b, 0, 0)),
        out_shape=jax.ShapeDtypeStruct((B, 1, C), jnp.float32),
    )(*([x] * S))

    idx = pl.pallas_call(
        functools.partial(_rank_body, B=B, C=C, k=k, HW=HW),
        in_specs=[
            pl.BlockSpec(memory_space=pltpu.SMEM),
            pl.BlockSpec((B, 1, C), lambda: (0, 0, 0)),
        ],
        out_specs=pl.BlockSpec((B, k), lambda: (0, 0)),
        out_shape=jax.ShapeDtypeStruct((B, k), jnp.int32),
    )(conv_w, sums3)

    U = 8
    out = pl.pallas_call(
        functools.partial(_gather_body, U=U),
        grid_spec=pltpu.PrefetchScalarGridSpec(
            num_scalar_prefetch=1,
            grid=(B, k // U),
            in_specs=[
                pl.BlockSpec((1, 1, H, W), functools.partial(
                    lambda u, b, r, idx: (b, idx[b, r * U + u], 0, 0), u))
                for u in range(U)
            ],
            out_specs=pl.BlockSpec((1, U, H, W), lambda b, r, idx: (b, r, 0, 0)),
        ),
        out_shape=jax.ShapeDtypeStruct((B, k, H, W), jnp.float32),
    )(idx, *([x] * U))
    return out


# speedup vs baseline: 1.3092x; 1.3092x over previous
import functools

import jax
import jax.numpy as jnp
from jax.experimental import pallas as pl
from jax.experimental.pallas import tpu as pltpu


def _pool_body(*refs, CB, S):
    x_refs, o_ref = refs[:S], refs[S]
    cb = pl.program_id(1)
    acc = jnp.sum(x_refs[0][...], axis=(0, 2, 3))
    for q in range(1, S):
        acc = acc + jnp.sum(x_refs[q][...], axis=(0, 2, 3))
    o_ref[0, 0, pl.ds(cb * CB, CB)] = acc


def _rank_body(w_ref, sums_ref, idx_ref, *, B, C, k, HW):
    w0 = w_ref[0]
    w1 = w_ref[1]
    w2 = w_ref[2]
    pooled = sums_ref[:, 0, :] / jnp.float32(HW)
    zero = jnp.zeros((B, 1), jnp.float32)
    left = jnp.concatenate([zero, pooled[:, :-1]], axis=1)
    right = jnp.concatenate([pooled[:, 1:], zero], axis=1)
    conv = w0 * left + w1 * pooled + w2 * right
    s = jax.nn.sigmoid(conv)
    ii = jax.lax.broadcasted_iota(jnp.int32, (C, C), 0)
    jj = jax.lax.broadcasted_iota(jnp.int32, (C, C), 1)
    rr = jax.lax.broadcasted_iota(jnp.int32, (C, k), 1)
    ic = jax.lax.broadcasted_iota(jnp.int32, (C, k), 0)
    for b in range(B):
        u = s[b].reshape(C, 1)
        v = s[b].reshape(1, C)
        before = (v > u) | ((v == u) & (jj < ii))
        rank = jnp.sum(before.astype(jnp.int32), axis=1, keepdims=True)
        sel = (rank == rr).astype(jnp.int32)
        idx_ref[b, :] = jnp.sum(ic * sel, axis=0)


def _gather_body(*refs, U):
    x_refs, o_ref = refs[1:1 + U], refs[1 + U]
    for u in range(U):
        o_ref[0, u] = x_refs[u][0, 0]


@jax.jit
def kernel(x, conv_w):
    B, C, H, W = x.shape
    k = int(C * 0.5)
    HW = H * W

    CB = 128
    NCB = C // CB
    S = 4
    HS = H // S
    sums3 = pl.pallas_call(
        functools.partial(_pool_body, CB=CB, S=S),
        grid=(B, NCB),
        in_specs=[
            pl.BlockSpec((1, CB, HS, W), functools.partial(
                lambda q, b, cb: (b, cb, q, 0), q))
            for q in range(S)
        ],
        out_specs=pl.BlockSpec((1, 1, C), lambda b, cb: (b, 0, 0)),
        out_shape=jax.ShapeDtypeStruct((B, 1, C), jnp.float32),
    )(*([x] * S))

    idx = pl.pallas_call(
        functools.partial(_rank_body, B=B, C=C, k=k, HW=HW),
        in_specs=[
            pl.BlockSpec(memory_space=pltpu.SMEM),
            pl.BlockSpec((B, 1, C), lambda: (0, 0, 0)),
        ],
        out_specs=pl.BlockSpec((B, k), lambda: (0, 0)),
        out_shape=jax.ShapeDtypeStruct((B, k), jnp.int32),
    )(conv_w, sums3)

    return sums3
    U = 8
    out = pl.pallas_call(
        functools.partial(_gather_body, U=U),
        grid_spec=pltpu.PrefetchScalarGridSpec(
            num_scalar_prefetch=1,
            grid=(B, k // U),
            in_specs=[
                pl.BlockSpec((1, 1, H, W), functools.partial(
                    lambda u, b, r, idx: (b, idx[b, r * U + u], 0, 0), u))
                for u in range(U)
            ],
            out_specs=pl.BlockSpec((1, U, H, W), lambda b, r, idx: (b, r, 0, 0)),
        ),
        out_shape=jax.ShapeDtypeStruct((B, k, H, W), jnp.float32),
    )(idx, *([x] * U))
    return out


# speedup vs baseline: 1.3141x; 1.0038x over previous
import functools

import jax
import jax.numpy as jnp
from jax.experimental import pallas as pl
from jax.experimental.pallas import tpu as pltpu


def _pool_body(*refs, S):
    x_refs, o_ref = refs[:S], refs[S]
    parts = [jnp.sum(x_refs[q][...], axis=(0, 2, 3)) for q in range(S)]
    o_ref[0, 0, 0, :] = jnp.concatenate(parts)


@jax.jit
def kernel(x, conv_w):
    B, C, H, W = x.shape
    CB = 128
    NCB = C // CB
    S = 4
    CBS = CB // S
    sums4 = pl.pallas_call(
        functools.partial(_pool_body, S=S),
        grid=(B, NCB),
        in_specs=[
            pl.BlockSpec((1, CBS, H, W), functools.partial(
                lambda q, b, cb: (b, cb * S + q, 0, 0), q))
            for q in range(S)
        ],
        out_specs=pl.BlockSpec((1, 1, 1, CB), lambda b, cb: (b, cb, 0, 0)),
        out_shape=jax.ShapeDtypeStruct((B, NCB, 1, CB), jnp.float32),
    )(*([x] * S))
    return sums4


# speedup vs baseline: 1.3152x; 1.0008x over previous
import functools

import jax
import jax.numpy as jnp
from jax.experimental import pallas as pl
from jax.experimental.pallas import tpu as pltpu


def _pool_body(*refs, S):
    x_refs, o_ref = refs[:S], refs[S]
    parts = [jnp.sum(x_refs[q][...], axis=(0, 2, 3)) for q in range(S)]
    o_ref[0, 0, 0, :] = jnp.concatenate(parts)


@jax.jit
def kernel(x, conv_w):
    B, C, H, W = x.shape
    CB = 32
    NCB = C // CB
    S = 4
    CBS = CB // S
    sums4 = pl.pallas_call(
        functools.partial(_pool_body, S=S),
        grid=(B, NCB),
        in_specs=[
            pl.BlockSpec((1, CBS, H, W), functools.partial(
                lambda q, b, cb: (b, cb * S + q, 0, 0), q))
            for q in range(S)
        ],
        out_specs=pl.BlockSpec((1, 1, 1, CB), lambda b, cb: (b, cb, 0, 0)),
        out_shape=jax.ShapeDtypeStruct((B, NCB, 1, CB), jnp.float32),
    )(*([x] * S))
    return sums4
